# baseline (speedup 1.0000x reference)
import functools

import jax
import jax.numpy as jnp
from jax import lax
from jax.experimental import pallas as pl
from jax.experimental.pallas import tpu as pltpu
from jax.experimental.pallas import tpu_sc as plsc

_NUM_CORES = 2
_NUM_SUBCORES = 16
_NW = _NUM_CORES * _NUM_SUBCORES
_CHUNK = 64


def _gather(labels, log_probs):
    (b,) = labels.shape
    _, d = log_probs.shape
    b_per_w = b // _NW
    n_ch = b_per_w // _CHUNK
    mesh = plsc.VectorSubcoreMesh(core_axis_name="c", subcore_axis_name="s")

    @functools.partial(
        pl.kernel,
        mesh=mesh,
        out_type=jax.ShapeDtypeStruct((b, d), jnp.float32),
        scratch_types=[
            pltpu.VMEM((n_ch * _CHUNK,), jnp.int32),
            pltpu.VMEM((n_ch, _CHUNK, d), jnp.float32),
            pltpu.SemaphoreType.DMA((n_ch,)),
            pltpu.SemaphoreType.DMA((n_ch,)),
            pltpu.SemaphoreType.DMA((n_ch,)),
        ],
    )
    def body(labels_hbm, table_hbm, out_hbm, idx_v, rows_v, lsem, gsem, ssem):
        wid = lax.axis_index("s") * _NUM_CORES + lax.axis_index("c")
        base = wid * b_per_w
        loads = [
            pltpu.async_copy(
                labels_hbm.at[pl.ds(base + j * _CHUNK, _CHUNK)],
                idx_v.at[pl.ds(j * _CHUNK, _CHUNK)],
                lsem.at[j],
            )
            for j in range(n_ch)
        ]
        gathers = []
        for j in range(n_ch):
            loads[j].wait()
            gathers.append(
                pltpu.async_copy(
                    table_hbm.at[idx_v.at[pl.ds(j * _CHUNK, _CHUNK)]],
                    rows_v.at[j],
                    gsem.at[j],
                )
            )
        stores = []
        for j in range(n_ch):
            gathers[j].wait()
            stores.append(
                pltpu.async_copy(
                    rows_v.at[j],
                    out_hbm.at[pl.ds(base + j * _CHUNK, _CHUNK)],
                    ssem.at[j],
                )
            )
        for s in stores:
            s.wait()

    return body(labels, log_probs)


def kernel(labels, log_probs):
    return _gather(labels.astype(jnp.int32), log_probs)


# speedup vs baseline: 1.0371x; 1.0371x over previous
import functools

import jax
import jax.numpy as jnp
from jax import lax
from jax.experimental import pallas as pl
from jax.experimental.pallas import tpu as pltpu
from jax.experimental.pallas import tpu_sc as plsc

_NUM_CORES = 2
_NUM_SUBCORES = 16
_NW = _NUM_CORES * _NUM_SUBCORES
_CHUNK = 256


def _gather(labels, log_probs):
    (b,) = labels.shape
    _, d = log_probs.shape
    b_per_w = b // _NW
    n_ch = b_per_w // _CHUNK
    mesh = plsc.VectorSubcoreMesh(core_axis_name="c", subcore_axis_name="s")

    @functools.partial(
        pl.kernel,
        mesh=mesh,
        out_type=jax.ShapeDtypeStruct((b, d), jnp.float32),
        scratch_types=[
            pltpu.VMEM((n_ch * _CHUNK,), jnp.int32),
            pltpu.VMEM((n_ch, _CHUNK, d), jnp.float32),
            pltpu.SemaphoreType.DMA((n_ch,)),
            pltpu.SemaphoreType.DMA((n_ch,)),
            pltpu.SemaphoreType.DMA((n_ch,)),
        ],
    )
    def body(labels_hbm, table_hbm, out_hbm, idx_v, rows_v, lsem, gsem, ssem):
        wid = lax.axis_index("s") * _NUM_CORES + lax.axis_index("c")
        base = wid * b_per_w
        loads = [
            pltpu.async_copy(
                labels_hbm.at[pl.ds(base + j * _CHUNK, _CHUNK)],
                idx_v.at[pl.ds(j * _CHUNK, _CHUNK)],
                lsem.at[j],
            )
            for j in range(n_ch)
        ]
        gathers = []
        for j in range(n_ch):
            loads[j].wait()
            gathers.append(
                pltpu.async_copy(
                    table_hbm.at[idx_v.at[pl.ds(j * _CHUNK, _CHUNK)]],
                    rows_v.at[j],
                    gsem.at[j],
                )
            )
        stores = []
        for j in range(n_ch):
            gathers[j].wait()
            stores.append(
                pltpu.async_copy(
                    rows_v.at[j],
                    out_hbm.at[pl.ds(base + j * _CHUNK, _CHUNK)],
                    ssem.at[j],
                )
            )
        for s in stores:
            s.wait()

    return body(labels, log_probs)


def kernel(labels, log_probs):
    return _gather(labels.astype(jnp.int32), log_probs)


# speedup vs baseline: 1.0391x; 1.0019x over previous
import functools

import jax
import jax.numpy as jnp
from jax import lax
from jax.experimental import pallas as pl
from jax.experimental.pallas import tpu as pltpu
from jax.experimental.pallas import tpu_sc as plsc

_NUM_CORES = 2
_NUM_SUBCORES = 16
_NW = _NUM_CORES * _NUM_SUBCORES
_CHUNK = 512


def _gather(labels, log_probs):
    (b,) = labels.shape
    _, d = log_probs.shape
    b_per_w = b // _NW
    n_ch = b_per_w // _CHUNK
    mesh = plsc.VectorSubcoreMesh(core_axis_name="c", subcore_axis_name="s")

    @functools.partial(
        pl.kernel,
        mesh=mesh,
        out_type=jax.ShapeDtypeStruct((b, d), jnp.float32),
        scratch_types=[
            pltpu.VMEM((n_ch * _CHUNK,), jnp.int32),
            pltpu.VMEM((n_ch, _CHUNK, d), jnp.float32),
            pltpu.SemaphoreType.DMA((n_ch,)),
            pltpu.SemaphoreType.DMA((n_ch,)),
            pltpu.SemaphoreType.DMA((n_ch,)),
        ],
    )
    def body(labels_hbm, table_hbm, out_hbm, idx_v, rows_v, lsem, gsem, ssem):
        wid = lax.axis_index("s") * _NUM_CORES + lax.axis_index("c")
        base = wid * b_per_w
        loads = [
            pltpu.async_copy(
                labels_hbm.at[pl.ds(base + j * _CHUNK, _CHUNK)],
                idx_v.at[pl.ds(j * _CHUNK, _CHUNK)],
                lsem.at[j],
            )
            for j in range(n_ch)
        ]
        gathers = []
        for j in range(n_ch):
            loads[j].wait()
            gathers.append(
                pltpu.async_copy(
                    table_hbm.at[idx_v.at[pl.ds(j * _CHUNK, _CHUNK)]],
                    rows_v.at[j],
                    gsem.at[j],
                )
            )
        stores = []
        for j in range(n_ch):
            gathers[j].wait()
            stores.append(
                pltpu.async_copy(
                    rows_v.at[j],
                    out_hbm.at[pl.ds(base + j * _CHUNK, _CHUNK)],
                    ssem.at[j],
                )
            )
        for s in stores:
            s.wait()

    return body(labels, log_probs)


def kernel(labels, log_probs):
    return _gather(labels.astype(jnp.int32), log_probs)


# speedup vs baseline: 1.0403x; 1.0012x over previous
import functools

import jax
import jax.numpy as jnp
from jax import lax
from jax.experimental import pallas as pl
from jax.experimental.pallas import tpu as pltpu
from jax.experimental.pallas import tpu_sc as plsc

_NUM_CORES = 2
_NUM_SUBCORES = 16
_NW = _NUM_CORES * _NUM_SUBCORES
_CHUNK = 256


def _gather(labels, log_probs):
    (b,) = labels.shape
    _, d = log_probs.shape
    b_per_w = b // _NW
    n_ch = b_per_w // _CHUNK
    mesh = plsc.VectorSubcoreMesh(core_axis_name="c", subcore_axis_name="s")

    @functools.partial(
        pl.kernel,
        mesh=mesh,
        out_type=jax.ShapeDtypeStruct((b, d), jnp.float32),
        scratch_types=[
            pltpu.VMEM((n_ch * _CHUNK,), jnp.int32),
            pltpu.VMEM((n_ch, _CHUNK, d), jnp.float32),
            pltpu.SemaphoreType.DMA((n_ch,)),
            pltpu.SemaphoreType.DMA((n_ch,)),
            pltpu.SemaphoreType.DMA((n_ch,)),
        ],
    )
    def body(labels_hbm, table_hbm, out_hbm, idx_v, rows_v, lsem, gsem, ssem):
        wid = lax.axis_index("s") * _NUM_CORES + lax.axis_index("c")
        base = wid * b_per_w
        loads = [
            pltpu.async_copy(
                labels_hbm.at[pl.ds(base + j * _CHUNK, _CHUNK)],
                idx_v.at[pl.ds(j * _CHUNK, _CHUNK)],
                lsem.at[j],
            )
            for j in range(n_ch)
        ]
        gathers = []
        for j in range(n_ch):
            loads[j].wait()
            gathers.append(
                pltpu.async_copy(
                    table_hbm.at[idx_v.at[pl.ds(j * _CHUNK, _CHUNK)]],
                    rows_v.at[j],
                    gsem.at[j],
                )
            )
        stores = []
        for j in range(n_ch):
            gathers[j].wait()
            stores.append(
                pltpu.async_copy(
                    rows_v.at[j],
                    out_hbm.at[pl.ds(base + j * _CHUNK, _CHUNK)],
                    ssem.at[j],
                )
            )
        for s in stores:
            s.wait()

    return body(labels, log_probs)


def kernel(labels, log_probs):
    return _gather(labels.astype(jnp.int32), log_probs)
